# baseline (speedup 1.0000x reference)
import jax
import jax.numpy as jnp
from jax import lax
from jax.experimental import pallas as pl
from jax.experimental.pallas import tpu as pltpu
from jax.experimental.pallas import tpu_sc as plsc

NUM_SLOTS = 64
K_SC = 24
OUT_SIDE = 1024
NC = 2
NS = 16
LANES = 16
NW = NC * NS
ROWS = OUT_SIDE // NW
CPR = OUT_SIDE // LANES
BR = 256


def _sc_body(wb_hbm, param_hbm, out_hbm, wb_v, buf0, buf1, acc, sem0, sem1):
    wid = lax.axis_index("s") * NC + lax.axis_index("c")
    row0 = wid * ROWS
    pltpu.sync_copy(wb_hbm, wb_v)
    bufs = (buf0, buf1)
    sems = (sem0, sem1)

    zero = jnp.zeros((LANES,), jnp.float32)

    def zbody(r):
        for k in range(CPR):
            acc[r, pl.ds(k * LANES, LANES)] = zero
    plsc.parallel_loop(0, ROWS, 1)(zbody)

    for b in range(2):
        pltpu.async_copy(param_hbm.at[b, pl.ds(row0, ROWS), :],
                         bufs[b], sems[b])

    def pair_body(g, carry):
        for b in range(2):
            s = g * 2 + b
            pltpu.make_async_copy(
                param_hbm.at[0, pl.ds(row0, ROWS), :],
                bufs[b], sems[b]).wait()
            wv = wb_v[s, pl.ds(0, LANES)]
            buf = bufs[b]

            def rbody(r, buf=buf, wv=wv):
                for k in range(CPR):
                    sl = pl.ds(k * LANES, LANES)
                    plsc.addupdate(acc.at[r, sl], wv * buf[r, sl])
            plsc.parallel_loop(0, ROWS, 1)(rbody)

            @pl.when(s + 2 < K_SC)
            def _():
                pltpu.async_copy(
                    param_hbm.at[s + 2, pl.ds(row0, ROWS), :],
                    bufs[b], sems[b])
        return carry
    lax.fori_loop(0, K_SC // 2, pair_body, 0)

    pltpu.sync_copy(acc, out_hbm.at[pl.ds(row0, ROWS), :])


def _sc_partial(wb, parameter):
    run = pl.kernel(
        _sc_body,
        out_type=jax.ShapeDtypeStruct((OUT_SIDE, OUT_SIDE), jnp.float32),
        mesh=plsc.VectorSubcoreMesh(
            core_axis_name="c", subcore_axis_name="s"),
        compiler_params=pltpu.CompilerParams(use_tc_tiling_on_sc=True),
        scratch_types=[
            pltpu.VMEM((NUM_SLOTS, 128), jnp.float32),
            pltpu.VMEM((ROWS, OUT_SIDE), jnp.float32),
            pltpu.VMEM((ROWS, OUT_SIDE), jnp.float32),
            pltpu.VMEM((ROWS, OUT_SIDE), jnp.float32),
            pltpu.SemaphoreType.DMA,
            pltpu.SemaphoreType.DMA,
        ],
    )
    return run(wb, parameter)


T_TC = 8


def _tc_reduce_body(w_ref, p_ref, out_ref):
    s = pl.program_id(1)
    psum = w_ref[K_SC + s * T_TC] * p_ref[0]
    for t in range(1, T_TC):
        psum += w_ref[K_SC + s * T_TC + t] * p_ref[t]

    @pl.when(s == 0)
    def _():
        out_ref[...] = psum

    @pl.when(s > 0)
    def _():
        out_ref[...] += psum


def _tc_partial(w1d, parameter):
    nt = (NUM_SLOTS - K_SC) // T_TC
    return pl.pallas_call(
        _tc_reduce_body,
        grid=(OUT_SIDE // BR, nt),
        in_specs=[
            pl.BlockSpec(memory_space=pltpu.SMEM),
            pl.BlockSpec((T_TC, BR, OUT_SIDE),
                         lambda i, s: (s + K_SC // T_TC, i, 0)),
        ],
        out_specs=pl.BlockSpec((BR, OUT_SIDE), lambda i, s: (i, 0)),
        out_shape=jax.ShapeDtypeStruct((OUT_SIDE, OUT_SIDE), jnp.float32),
        compiler_params=pltpu.CompilerParams(
            dimension_semantics=("parallel", "arbitrary")),
    )(w1d, parameter)


def _add_body(a_ref, b_ref, o_ref):
    o_ref[...] = a_ref[...] + b_ref[...]


BA = 128


def _combine(a, b):
    return pl.pallas_call(
        _add_body,
        grid=(OUT_SIDE // BA,),
        in_specs=[
            pl.BlockSpec((BA, OUT_SIDE), lambda i: (i, 0)),
            pl.BlockSpec((BA, OUT_SIDE), lambda i: (i, 0)),
        ],
        out_specs=pl.BlockSpec((BA, OUT_SIDE), lambda i: (i, 0)),
        out_shape=jax.ShapeDtypeStruct((OUT_SIDE, OUT_SIDE), jnp.float32),
        input_output_aliases={0: 0},
    )(a, b)


def kernel(superposition_weights, parameter):
    wb = jnp.broadcast_to(
        superposition_weights[:, None], (NUM_SLOTS, 128))
    part_sc = _sc_partial(wb, parameter)
    part_tc = _tc_partial(superposition_weights, parameter)
    return _combine(part_sc, part_tc)


# speedup vs baseline: 1.0346x; 1.0346x over previous
import jax
import jax.numpy as jnp
from jax import lax
from jax.experimental import pallas as pl
from jax.experimental.pallas import tpu as pltpu
from jax.experimental.pallas import tpu_sc as plsc

NUM_SLOTS = 64
K_SC = 24
OUT_SIDE = 1024
NC = 2
NS = 16
LANES = 16
NW = NC * NS
ROWS = OUT_SIDE // NW
CPR = OUT_SIDE // LANES
BR = 256


def _sc_body(w_hbm, param_hbm, out_hbm, w_v, buf0, buf1, acc, sem0, sem1):
    wid = lax.axis_index("s") * NC + lax.axis_index("c")
    row0 = wid * ROWS
    pltpu.sync_copy(w_hbm, w_v)
    bufs = (buf0, buf1)
    sems = (sem0, sem1)

    zero = jnp.zeros((LANES,), jnp.float32)

    def zbody(r):
        for k in range(CPR):
            acc[r, pl.ds(k * LANES, LANES)] = zero
    plsc.parallel_loop(0, ROWS, 1)(zbody)

    for b in range(2):
        pltpu.async_copy(param_hbm.at[b, pl.ds(row0, ROWS), :],
                         bufs[b], sems[b])

    def pair_body(g, carry):
        for b in range(2):
            s = g * 2 + b
            pltpu.make_async_copy(
                param_hbm.at[0, pl.ds(row0, ROWS), :],
                bufs[b], sems[b]).wait()
            group = w_v[pl.ds((s // LANES) * LANES, LANES)]
            idx = jnp.full((LANES,), s % LANES, jnp.int32)
            wv = lax.gather(
                group, idx[:, None],
                lax.GatherDimensionNumbers(
                    offset_dims=(), collapsed_slice_dims=(0,),
                    start_index_map=(0,)),
                slice_sizes=(1,),
                mode=lax.GatherScatterMode.PROMISE_IN_BOUNDS)
            buf = bufs[b]

            def rbody(r, buf=buf, wv=wv):
                for k in range(CPR):
                    sl = pl.ds(k * LANES, LANES)
                    plsc.addupdate(acc.at[r, sl], wv * buf[r, sl])
            plsc.parallel_loop(0, ROWS, 1)(rbody)

            @pl.when(s + 2 < K_SC)
            def _():
                pltpu.async_copy(
                    param_hbm.at[s + 2, pl.ds(row0, ROWS), :],
                    bufs[b], sems[b])
        return carry
    lax.fori_loop(0, K_SC // 2, pair_body, 0)

    pltpu.sync_copy(acc, out_hbm.at[pl.ds(row0, ROWS), :])


def _sc_partial(w1d, parameter):
    run = pl.kernel(
        _sc_body,
        out_type=jax.ShapeDtypeStruct((OUT_SIDE, OUT_SIDE), jnp.float32),
        mesh=plsc.VectorSubcoreMesh(
            core_axis_name="c", subcore_axis_name="s"),
        compiler_params=pltpu.CompilerParams(use_tc_tiling_on_sc=True),
        scratch_types=[
            pltpu.VMEM((NUM_SLOTS,), jnp.float32),
            pltpu.VMEM((ROWS, OUT_SIDE), jnp.float32),
            pltpu.VMEM((ROWS, OUT_SIDE), jnp.float32),
            pltpu.VMEM((ROWS, OUT_SIDE), jnp.float32),
            pltpu.SemaphoreType.DMA,
            pltpu.SemaphoreType.DMA,
        ],
    )
    return run(w1d, parameter)


T_TC = 8


def _tc_reduce_body(w_ref, p_ref, out_ref):
    s = pl.program_id(1)
    psum = w_ref[K_SC + s * T_TC] * p_ref[0]
    for t in range(1, T_TC):
        psum += w_ref[K_SC + s * T_TC + t] * p_ref[t]

    @pl.when(s == 0)
    def _():
        out_ref[...] = psum

    @pl.when(s > 0)
    def _():
        out_ref[...] += psum


def _tc_partial(w1d, parameter):
    nt = (NUM_SLOTS - K_SC) // T_TC
    return pl.pallas_call(
        _tc_reduce_body,
        grid=(OUT_SIDE // BR, nt),
        in_specs=[
            pl.BlockSpec(memory_space=pltpu.SMEM),
            pl.BlockSpec((T_TC, BR, OUT_SIDE),
                         lambda i, s: (s + K_SC // T_TC, i, 0)),
        ],
        out_specs=pl.BlockSpec((BR, OUT_SIDE), lambda i, s: (i, 0)),
        out_shape=jax.ShapeDtypeStruct((OUT_SIDE, OUT_SIDE), jnp.float32),
        compiler_params=pltpu.CompilerParams(
            dimension_semantics=("parallel", "arbitrary")),
    )(w1d, parameter)


def _add_body(a_ref, b_ref, o_ref):
    o_ref[...] = a_ref[...] + b_ref[...]


BA = 256


def _combine(a, b):
    return pl.pallas_call(
        _add_body,
        grid=(OUT_SIDE // BA,),
        in_specs=[
            pl.BlockSpec((BA, OUT_SIDE), lambda i: (i, 0)),
            pl.BlockSpec((BA, OUT_SIDE), lambda i: (i, 0)),
        ],
        out_specs=pl.BlockSpec((BA, OUT_SIDE), lambda i: (i, 0)),
        out_shape=jax.ShapeDtypeStruct((OUT_SIDE, OUT_SIDE), jnp.float32),
    )(a, b)


def kernel(superposition_weights, parameter):
    part_sc = _sc_partial(superposition_weights, parameter)
    part_tc = _tc_partial(superposition_weights, parameter)
    return _combine(part_sc, part_tc)
